# baseline (speedup 1.0000x reference)
import functools

import jax
import jax.numpy as jnp
from jax import lax
from jax.experimental import pallas as pl
from jax.experimental.pallas import tpu as pltpu
from jax.experimental.pallas import tpu_sc as plsc

_NC = 2
_NS = 16
_NW = _NC * _NS
_ROW = 128
_CHUNK = 128
_CPG = 2
_GPH = 2


def _make_gather(hist: int, batch: int, emb_dim: int):
    ipg = _CHUNK * _CPG
    cols = ipg * _GPH
    n_groups = hist * _GPH
    assert batch == _NW * cols and n_groups % 2 == 0
    per_row = _ROW // emb_dim
    mesh = plsc.VectorSubcoreMesh(core_axis_name="c", subcore_axis_name="s")

    n_btile = batch // 128
    btpg = ipg // 128

    @functools.partial(
        pl.kernel,
        out_type=jax.ShapeDtypeStruct(
            (hist, emb_dim // 8, n_btile, 8, 128), jnp.float32),
        mesh=mesh,
        compiler_params=pltpu.CompilerParams(use_tc_tiling_on_sc=False,
                                             needs_layout_passes=False),
        scratch_types=[
            pltpu.VMEM((hist, cols), jnp.int32),
            pltpu.VMEM((2 * _CPG, _CHUNK, _ROW), jnp.float32),
            pltpu.VMEM((2 * _CPG, _CHUNK), jnp.int32),
            pltpu.VMEM((2, emb_dim // 8, btpg, 8, 128), jnp.float32),
            pltpu.SemaphoreType.DMA,
            pltpu.SemaphoreType.DMA,
            pltpu.SemaphoreType.DMA,
            pltpu.SemaphoreType.DMA,
        ],
    )
    def gather_kernel(ids_hbm, table_hbm, out_hbm, idx_v, rows_v, qrow_v,
                      slab_v, gsem0, gsem1, wsem0, wsem1):
        wid = lax.axis_index("s") * _NC + lax.axis_index("c")
        gsems = (gsem0, gsem1)
        wsems = (wsem0, wsem1)
        col0 = pl.multiple_of(wid * cols, cols)

        pltpu.sync_copy(ids_hbm.at[:, pl.ds(col0, cols)], idx_v)

        iotav = lax.iota(jnp.int32, 16)
        item_vecs = [iotav + (lg * 16) for lg in range(8)]
        n_lg = _CHUNK // 16

        def idx_col(g, cb):
            return lax.bitwise_and(g, _GPH - 1) * ipg + cb * _CHUNK

        def fire_group(g, half, sem):
            h = lax.shift_right_logical(g, 1)
            for cb in range(_CPG):
                s = half * _CPG + cb
                c0 = idx_col(g, cb)
                for lg in range(n_lg):
                    iv = idx_v[h, pl.ds(c0 + lg * 16, 16)]
                    qrow_v[s, pl.ds(lg * 16, 16)] = lax.shift_right_logical(
                        iv, 2)
                pltpu.async_copy(table_hbm.at[qrow_v.at[s]], rows_v.at[s],
                                 sem)

        def drain_group(half, sem):
            for cb in range(_CPG):
                s = half * _CPG + cb
                pltpu.make_async_copy(table_hbm.at[qrow_v.at[s]],
                                      rows_v.at[s], sem).wait()

        def transpose_group(g, half):
            h = lax.shift_right_logical(g, 1)
            qoffs = []
            for cb in range(_CPG):
                c0 = idx_col(g, cb)
                for lg in range(n_lg):
                    iv = idx_v[h, pl.ds(c0 + lg * 16, 16)]
                    qoffs.append(lax.shift_left(
                        lax.bitwise_and(iv, per_row - 1), 5))

            @plsc.parallel_loop(0, emb_dim, unroll=4)
            def _(f):
                fr = lax.shift_right_logical(f, 3)
                fs = lax.bitwise_and(f, 7)
                for cb in range(_CPG):
                    s = half * _CPG + cb
                    for lg in range(n_lg):
                        vals = plsc.load_gather(
                            rows_v.at[s],
                            [item_vecs[lg], qoffs[cb * n_lg + lg] + f])
                        c = cb * _CHUNK + lg * 16
                        slab_v[half, fr, c // 128, fs,
                               pl.ds(c % 128, 16)] = vals

        def wb_copy(g, half, sem):
            h = lax.shift_right_logical(g, 1)
            bt0 = pl.multiple_of(
                wid * (cols // 128) + lax.bitwise_and(g, _GPH - 1) * btpg,
                btpg)
            return pltpu.make_async_copy(
                slab_v.at[half],
                out_hbm.at[h, :, pl.ds(bt0, btpg)], sem)

        fire_group(0, 0, gsem0)

        @pl.loop(0, n_groups, step=2)
        def _(g0):
            for hh in (0, 1):
                g = g0 + hh
                @pl.when(g >= 2)
                def _():
                    wb_copy(g - 2, hh, wsems[hh]).wait()

                @pl.when(g + 1 < n_groups)
                def _():
                    fire_group(g + 1, 1 - hh, gsems[1 - hh])

                drain_group(hh, gsems[hh])
                transpose_group(g, hh)
                wb = wb_copy(g, hh, wsems[hh])
                wb.start()

        wb_copy(n_groups - 2, 0, wsem0).wait()
        wb_copy(n_groups - 1, 1, wsem1).wait()

    return gather_kernel


def kernel(item_ids, table):
    batch, hist = item_ids.shape
    n_vocab, emb_dim = table.shape
    assert batch % (_NW * _CHUNK * _CPG * _GPH) == 0 and hist % 2 == 0
    assert (n_vocab * emb_dim) % _ROW == 0 and _ROW % emb_dim == 0
    table_view = table.reshape(n_vocab * emb_dim // _ROW, _ROW)
    out = _make_gather(hist, batch, emb_dim)(item_ids.T, table_view)
    return out.transpose(2, 4, 0, 1, 3).reshape(batch, hist, emb_dim)


# speedup vs baseline: 1.0214x; 1.0214x over previous
import functools

import jax
import jax.numpy as jnp
from jax import lax
from jax.experimental import pallas as pl
from jax.experimental.pallas import tpu as pltpu
from jax.experimental.pallas import tpu_sc as plsc

_NC = 2
_NS = 16
_NW = _NC * _NS
_ROW = 128
_CHUNK = 64
_CPG = 4
_GPH = 2


def _make_gather(hist: int, batch: int, emb_dim: int):
    ipg = _CHUNK * _CPG
    cols = ipg * _GPH
    n_groups = hist * _GPH
    assert batch == _NW * cols and n_groups % 2 == 0
    per_row = _ROW // emb_dim
    mesh = plsc.VectorSubcoreMesh(core_axis_name="c", subcore_axis_name="s")

    n_btile = batch // 128
    btpg = ipg // 128

    @functools.partial(
        pl.kernel,
        out_type=jax.ShapeDtypeStruct(
            (hist, emb_dim // 8, n_btile, 8, 128), jnp.float32),
        mesh=mesh,
        compiler_params=pltpu.CompilerParams(use_tc_tiling_on_sc=True,
                                             needs_layout_passes=False),
        scratch_types=[
            pltpu.VMEM((hist, cols), jnp.int32),
            pltpu.VMEM((2 * _CPG, _CHUNK, _ROW), jnp.float32),
            pltpu.VMEM((2 * _CPG, _CHUNK), jnp.int32),
            pltpu.VMEM((2, emb_dim // 8, btpg, 8, 128), jnp.float32),
            pltpu.SemaphoreType.DMA,
            pltpu.SemaphoreType.DMA,
            pltpu.SemaphoreType.DMA,
            pltpu.SemaphoreType.DMA,
        ],
    )
    def gather_kernel(ids_hbm, table_hbm, out_hbm, idx_v, rows_v, qrow_v,
                      slab_v, gsem0, gsem1, wsem0, wsem1):
        wid = lax.axis_index("s") * _NC + lax.axis_index("c")
        gsems = (gsem0, gsem1)
        wsems = (wsem0, wsem1)
        col0 = pl.multiple_of(wid * cols, cols)

        pltpu.sync_copy(ids_hbm.at[:, pl.ds(col0, cols)], idx_v)

        iotav = lax.iota(jnp.int32, 16)
        item_vecs = [iotav + (lg * 16) for lg in range(8)]
        n_lg = _CHUNK // 16

        def idx_col(g, cb):
            return lax.bitwise_and(g, _GPH - 1) * ipg + cb * _CHUNK

        def fire_group(g, half, sem):
            h = lax.shift_right_logical(g, 1)
            for cb in range(_CPG):
                s = half * _CPG + cb
                c0 = idx_col(g, cb)
                for lg in range(n_lg):
                    iv = idx_v[h, pl.ds(c0 + lg * 16, 16)]
                    qrow_v[s, pl.ds(lg * 16, 16)] = lax.shift_right_logical(
                        iv, 2)
                pltpu.async_copy(table_hbm.at[qrow_v.at[s]], rows_v.at[s],
                                 sem)

        def drain_group(half, sem):
            for cb in range(_CPG):
                s = half * _CPG + cb
                pltpu.make_async_copy(table_hbm.at[qrow_v.at[s]],
                                      rows_v.at[s], sem).wait()

        def transpose_group(g, half):
            h = lax.shift_right_logical(g, 1)
            qoffs = []
            for cb in range(_CPG):
                c0 = idx_col(g, cb)
                for lg in range(n_lg):
                    iv = idx_v[h, pl.ds(c0 + lg * 16, 16)]
                    qoffs.append(lax.shift_left(
                        lax.bitwise_and(iv, per_row - 1), 5))

            @plsc.parallel_loop(0, emb_dim, unroll=2)
            def _(f):
                fr = lax.shift_right_logical(f, 3)
                fs = lax.bitwise_and(f, 7)
                for cb in range(_CPG):
                    s = half * _CPG + cb
                    for lg in range(n_lg):
                        vals = plsc.load_gather(
                            rows_v.at[s],
                            [item_vecs[lg], qoffs[cb * n_lg + lg] + f])
                        c = cb * _CHUNK + lg * 16
                        slab_v[half, fr, c // 128, fs,
                               pl.ds(c % 128, 16)] = vals

        def wb_copy(g, half, sem):
            h = lax.shift_right_logical(g, 1)
            bt0 = pl.multiple_of(
                wid * (cols // 128) + lax.bitwise_and(g, _GPH - 1) * btpg,
                btpg)
            return pltpu.make_async_copy(
                slab_v.at[half],
                out_hbm.at[h, :, pl.ds(bt0, btpg)], sem)

        fire_group(0, 0, gsem0)

        @pl.loop(0, n_groups, step=2)
        def _(g0):
            for hh in (0, 1):
                g = g0 + hh
                @pl.when(g >= 2)
                def _():
                    wb_copy(g - 2, hh, wsems[hh]).wait()

                @pl.when(g + 1 < n_groups)
                def _():
                    fire_group(g + 1, 1 - hh, gsems[1 - hh])

                drain_group(hh, gsems[hh])
                transpose_group(g, hh)
                wb = wb_copy(g, hh, wsems[hh])
                wb.start()

        wb_copy(n_groups - 2, 0, wsem0).wait()
        wb_copy(n_groups - 1, 1, wsem1).wait()

    return gather_kernel


def kernel(item_ids, table):
    batch, hist = item_ids.shape
    n_vocab, emb_dim = table.shape
    assert batch % (_NW * _CHUNK * _CPG * _GPH) == 0 and hist % 2 == 0
    assert (n_vocab * emb_dim) % _ROW == 0 and _ROW % emb_dim == 0
    table_view = table.reshape(n_vocab * emb_dim // _ROW, _ROW)
    out = _make_gather(hist, batch, emb_dim)(item_ids.T, table_view)
    return out.transpose(2, 4, 0, 1, 3).reshape(batch, hist, emb_dim)
